# baseline (speedup 1.0000x reference)
import functools

import jax
import jax.numpy as jnp
from jax import lax
from jax.experimental import pallas as pl
from jax.experimental.pallas import tpu as pltpu
from jax.experimental.pallas import tpu_sc as plsc

EMB = 64
HID = 400
OUT = 1000
B, T, W = 1024, 50, 20

NW = 32
BLK = 32
LANES = 16
NCHUNK = EMB // LANES

TCH = 10
NCH = T // TCH


def _sc_gather_sum(x, embedding, t0, tn):
    rows = tn * B
    nblk = tn
    assert nblk % 2 == 0
    mesh = plsc.VectorSubcoreMesh(core_axis_name="c", subcore_axis_name="s")

    @functools.partial(
        pl.kernel,
        out_type=jax.ShapeDtypeStruct((rows, EMB), jnp.float32),
        mesh=mesh,
        scratch_types=[
            pltpu.VMEM((BLK, W), jnp.int32),
            pltpu.VMEM((BLK, W), jnp.int32),
            pltpu.VMEM((BLK * W,), jnp.int32),
            pltpu.VMEM((BLK * W,), jnp.int32),
            pltpu.VMEM((BLK * W, EMB), jnp.float32),
            pltpu.VMEM((BLK * W, EMB), jnp.float32),
            pltpu.VMEM((BLK, EMB), jnp.float32),
            pltpu.SemaphoreType.DMA,
            pltpu.SemaphoreType.DMA,
        ],
        compiler_params=pltpu.CompilerParams(use_tc_tiling_on_sc=False),
    )
    def gather_sum(x_hbm, emb_hbm, out_hbm, st0, st1, idx0, idx1,
                   rows0, rows1, out_v, sem0, sem1):
        wid = lax.axis_index("s") * 2 + lax.axis_index("c")
        b0 = wid * BLK
        bufs = ((st0, idx0, rows0, sem0), (st1, idx1, rows1, sem1))

        def start(blk, buf):
            st_v, idx_v, rows_v, sem = buf
            pltpu.sync_copy(x_hbm.at[pl.ds(b0, BLK), t0 + blk], st_v)

            @pl.loop(0, BLK)
            def _(k):
                lo = st_v[k, pl.ds(0, LANES)]
                hi = st_v[k, pl.ds(W - LANES, LANES)]
                idx_v[pl.ds(k * W, LANES)] = lo
                idx_v[pl.ds(k * W + W - LANES, LANES)] = hi

            pltpu.make_async_copy(emb_hbm.at[idx_v], rows_v, sem).start()

        def finish(blk, buf):
            st_v, idx_v, rows_v, sem = buf
            pltpu.make_async_copy(emb_hbm.at[idx_v], rows_v, sem).wait()

            @pl.loop(0, BLK)
            def _(k):
                row = k * W
                accs = [rows_v[row, pl.ds(c * LANES, LANES)]
                        for c in range(NCHUNK)]
                for w in range(1, W):
                    accs = [accs[c] + rows_v[row + w, pl.ds(c * LANES, LANES)]
                            for c in range(NCHUNK)]
                for c in range(NCHUNK):
                    out_v[k, pl.ds(c * LANES, LANES)] = accs[c]

            pltpu.sync_copy(out_v, out_hbm.at[pl.ds(blk * B + b0, BLK)])

        start(0, bufs[0])
        start(1, bufs[1])

        @pl.loop(0, nblk - 2, step=2)
        def _(g):
            for b in range(2):
                finish(g + b, bufs[b])
                start(g + b + 2, bufs[b])

        for b in range(2):
            finish(nblk - 2 + b, bufs[b])

    return gather_sum(x, embedding)


K1 = EMB + HID + 1


def _lstm_chunk_body(s_ref, h_ref, c_ref, wi, wf, wg, wo,
                     ho_ref, co_ref, xcat_ref, c_scr):
    xcat_ref[:, EMB:EMB + HID] = h_ref[...].astype(jnp.bfloat16)
    xcat_ref[:, EMB + HID:] = jnp.ones((B, 1), jnp.bfloat16)
    c_scr[...] = c_ref[...]

    def step(t, _):
        xcat_ref[:, :EMB] = s_ref[t].astype(jnp.bfloat16)
        xb = xcat_ref[...]
        dot = lambda a, b: jnp.dot(a, b, preferred_element_type=jnp.float32)
        sig = lambda z: 0.5 * jnp.tanh(0.5 * z) + 0.5
        ig = sig(dot(xb, wi[...]))
        fg = sig(dot(xb, wf[...]))
        gg = jnp.tanh(dot(xb, wg[...]))
        og = sig(dot(xb, wo[...]))
        c2 = fg * c_scr[...] + ig * gg
        c_scr[...] = c2
        xcat_ref[:, EMB:EMB + HID] = (og * jnp.tanh(c2)).astype(jnp.bfloat16)
        return 0

    lax.fori_loop(0, TCH, step, 0)
    ho_ref[...] = xcat_ref[:, EMB:EMB + HID].astype(jnp.float32)
    co_ref[...] = c_scr[...]


def _lstm_chunk(s, h, c, ws):
    return pl.pallas_call(
        _lstm_chunk_body,
        out_shape=(jax.ShapeDtypeStruct((B, HID), jnp.float32),
                   jax.ShapeDtypeStruct((B, HID), jnp.float32)),
        scratch_shapes=[pltpu.VMEM((B, K1), jnp.bfloat16),
                        pltpu.VMEM((B, HID), jnp.float32)],
    )(s, h, c, *ws)


def _dense_softmax_body(h_ref, wd, bd, out_ref):
    logits = jnp.dot(h_ref[...], wd[...],
                     preferred_element_type=jnp.float32) + bd[...]
    m = jnp.max(logits, axis=-1, keepdims=True)
    e = jnp.exp(logits - m)
    out_ref[...] = e / jnp.sum(e, axis=-1, keepdims=True)


def kernel(x, embedding, W_lstm, U_lstm, b_lstm, W_dense, b_dense):
    wcat = jnp.concatenate(
        [W_lstm, U_lstm, b_lstm.reshape(1, 4 * HID)], axis=0
    ).astype(jnp.bfloat16)
    ws = [wcat[:, i * HID:(i + 1) * HID] for i in range(4)]

    h = jnp.zeros((B, HID), jnp.float32)
    c = jnp.zeros((B, HID), jnp.float32)
    s_chunks = {}
    for k in range(2):
        s_chunks[k] = _sc_gather_sum(x, embedding, k * TCH, TCH)
    for k in range(NCH):
        h, c = _lstm_chunk(s_chunks[k].reshape(TCH, B, EMB), h, c, ws)
        nxt = k + 2
        if nxt < NCH:
            xdep, _ = lax.optimization_barrier((x, h))
            s_chunks[nxt] = _sc_gather_sum(xdep, embedding, nxt * TCH, TCH)

    return pl.pallas_call(
        _dense_softmax_body,
        out_shape=jax.ShapeDtypeStruct((B, OUT), jnp.float32),
    )(h, W_dense, b_dense.reshape(1, OUT))


# speedup vs baseline: 1.4282x; 1.4282x over previous
import functools

import jax
import jax.numpy as jnp
from jax import lax
from jax.experimental import pallas as pl
from jax.experimental.pallas import tpu as pltpu
from jax.experimental.pallas import tpu_sc as plsc

EMB = 64
HID = 400
OUT = 1000
B, T, W = 1024, 50, 20

NW = 32
BLK = 32
LANES = 16
NCHUNK = EMB // LANES

TCH = 10
NCH = T // TCH


def _sc_gather_sum(x, embedding, dep, t0, tn):
    rows = tn * B
    nblk = tn
    assert nblk % 2 == 0
    mesh = plsc.VectorSubcoreMesh(core_axis_name="c", subcore_axis_name="s")

    @functools.partial(
        pl.kernel,
        out_type=jax.ShapeDtypeStruct((rows, EMB), jnp.float32),
        mesh=mesh,
        scratch_types=[
            pltpu.VMEM((BLK, W), jnp.int32),
            pltpu.VMEM((BLK, W), jnp.int32),
            pltpu.VMEM((BLK * W,), jnp.int32),
            pltpu.VMEM((BLK * W,), jnp.int32),
            pltpu.VMEM((BLK * W, EMB), jnp.float32),
            pltpu.VMEM((BLK * W, EMB), jnp.float32),
            pltpu.VMEM((BLK, EMB), jnp.float32),
            pltpu.SemaphoreType.DMA,
            pltpu.SemaphoreType.DMA,
        ],
        compiler_params=pltpu.CompilerParams(use_tc_tiling_on_sc=False),
    )
    def gather_sum(x_hbm, emb_hbm, dep_hbm, out_hbm, st0, st1, idx0, idx1,
                   rows0, rows1, out_v, sem0, sem1):
        del dep_hbm
        wid = lax.axis_index("s") * 2 + lax.axis_index("c")
        b0 = wid * BLK
        bufs = ((st0, idx0, rows0, sem0), (st1, idx1, rows1, sem1))

        def start(blk, buf):
            st_v, idx_v, rows_v, sem = buf
            pltpu.sync_copy(x_hbm.at[pl.ds(b0, BLK), t0 + blk], st_v)

            @pl.loop(0, BLK)
            def _(k):
                lo = st_v[k, pl.ds(0, LANES)]
                hi = st_v[k, pl.ds(W - LANES, LANES)]
                idx_v[pl.ds(k * W, LANES)] = lo
                idx_v[pl.ds(k * W + W - LANES, LANES)] = hi

            pltpu.make_async_copy(emb_hbm.at[idx_v], rows_v, sem).start()

        def finish(blk, buf):
            st_v, idx_v, rows_v, sem = buf
            pltpu.make_async_copy(emb_hbm.at[idx_v], rows_v, sem).wait()

            @pl.loop(0, BLK)
            def _(k):
                row = k * W
                accs = [rows_v[row, pl.ds(c * LANES, LANES)]
                        for c in range(NCHUNK)]
                for w in range(1, W):
                    accs = [accs[c] + rows_v[row + w, pl.ds(c * LANES, LANES)]
                            for c in range(NCHUNK)]
                for c in range(NCHUNK):
                    out_v[k, pl.ds(c * LANES, LANES)] = accs[c]

            pltpu.sync_copy(out_v, out_hbm.at[pl.ds(blk * B + b0, BLK)])

        start(0, bufs[0])
        start(1, bufs[1])

        @pl.loop(0, nblk - 2, step=2)
        def _(g):
            for b in range(2):
                finish(g + b, bufs[b])
                start(g + b + 2, bufs[b])

        for b in range(2):
            finish(nblk - 2 + b, bufs[b])

    return gather_sum(x, embedding, dep)


K1 = EMB + HID + 1


def _lstm_chunk_body(s_ref, h_ref, c_ref, wi, wf, wg, wo,
                     ho_ref, co_ref, xcat_ref, c_scr):
    xcat_ref[:, EMB:EMB + HID] = h_ref[...].astype(jnp.bfloat16)
    xcat_ref[:, EMB + HID:] = jnp.ones((B, 1), jnp.bfloat16)
    c_scr[...] = c_ref[...]

    def step(t, _):
        xcat_ref[:, :EMB] = s_ref[t].astype(jnp.bfloat16)
        xb = xcat_ref[...]
        dot = lambda a, b: jnp.dot(a, b, preferred_element_type=jnp.float32)
        sig = lambda z: 0.5 * jnp.tanh(0.5 * z) + 0.5
        ig = sig(dot(xb, wi[...]))
        fg = sig(dot(xb, wf[...]))
        gg = jnp.tanh(dot(xb, wg[...]))
        og = sig(dot(xb, wo[...]))
        c2 = fg * c_scr[...] + ig * gg
        c_scr[...] = c2
        xcat_ref[:, EMB:EMB + HID] = (og * jnp.tanh(c2)).astype(jnp.bfloat16)
        return 0

    lax.fori_loop(0, TCH, step, 0)
    ho_ref[...] = xcat_ref[:, EMB:EMB + HID].astype(jnp.float32)
    co_ref[...] = c_scr[...]


def _lstm_chunk(s, h, c, ws):
    return pl.pallas_call(
        _lstm_chunk_body,
        out_shape=(jax.ShapeDtypeStruct((B, HID), jnp.float32),
                   jax.ShapeDtypeStruct((B, HID), jnp.float32)),
        scratch_shapes=[pltpu.VMEM((B, K1), jnp.bfloat16),
                        pltpu.VMEM((B, HID), jnp.float32)],
    )(s, h, c, *ws)


def _dense_softmax_body(h_ref, wd, bd, out_ref):
    logits = jnp.dot(h_ref[...], wd[...],
                     preferred_element_type=jnp.float32) + bd[...]
    m = jnp.max(logits, axis=-1, keepdims=True)
    e = jnp.exp(logits - m)
    out_ref[...] = e / jnp.sum(e, axis=-1, keepdims=True)


def kernel(x, embedding, W_lstm, U_lstm, b_lstm, W_dense, b_dense):
    wcat = jnp.concatenate(
        [W_lstm, U_lstm, b_lstm.reshape(1, 4 * HID)], axis=0
    ).astype(jnp.bfloat16)
    ws = [wcat[:, i * HID:(i + 1) * HID] for i in range(4)]

    h = jnp.zeros((B, HID), jnp.float32)
    c = jnp.zeros((B, HID), jnp.float32)
    zdep = jnp.zeros((8, 8), jnp.float32)
    s_chunks = {}
    for k in range(2):
        s_chunks[k] = _sc_gather_sum(x, embedding, zdep, k * TCH, TCH)
    for k in range(NCH):
        h, c = _lstm_chunk(s_chunks[k].reshape(TCH, B, EMB), h, c, ws)
        nxt = k + 2
        if nxt < NCH:
            s_chunks[nxt] = _sc_gather_sum(x, embedding, h[:8, :8],
                                           nxt * TCH, TCH)

    return pl.pallas_call(
        _dense_softmax_body,
        out_shape=jax.ShapeDtypeStruct((B, OUT), jnp.float32),
    )(h, W_dense, b_dense.reshape(1, OUT))


# speedup vs baseline: 1.5032x; 1.0525x over previous
import functools

import jax
import jax.numpy as jnp
from jax import lax
from jax.experimental import pallas as pl
from jax.experimental.pallas import tpu as pltpu
from jax.experimental.pallas import tpu_sc as plsc

EMB = 64
HID = 400
OUT = 1000
B, T, W = 1024, 50, 20

NW = 32
BLK = 32
LANES = 16
NCHUNK = EMB // LANES

CHS = (2, 12, 12, 12, 12)
NCH = len(CHS)


def _sc_gather_sum(xflat, embedding, dep, t0, tn):
    rows = tn * B
    rows_per_w = rows // NW
    nblk = rows_per_w // BLK
    assert nblk % 2 == 0
    mesh = plsc.VectorSubcoreMesh(core_axis_name="c", subcore_axis_name="s")

    @functools.partial(
        pl.kernel,
        out_type=jax.ShapeDtypeStruct((rows, EMB), jnp.float32),
        mesh=mesh,
        scratch_types=[
            pltpu.VMEM((BLK * W,), jnp.int32),
            pltpu.VMEM((BLK * W,), jnp.int32),
            pltpu.VMEM((BLK * W, EMB), jnp.float32),
            pltpu.VMEM((BLK * W, EMB), jnp.float32),
            pltpu.VMEM((BLK, EMB), jnp.float32),
            pltpu.SemaphoreType.DMA,
            pltpu.SemaphoreType.DMA,
        ],
        compiler_params=pltpu.CompilerParams(use_tc_tiling_on_sc=False),
    )
    def gather_sum(x_hbm, emb_hbm, dep_hbm, out_hbm, idx0, idx1,
                   rows0, rows1, out_v, sem0, sem1):
        del dep_hbm
        wid = lax.axis_index("s") * 2 + lax.axis_index("c")
        base0 = wid * rows_per_w
        bufs = ((idx0, rows0, sem0), (idx1, rows1, sem1))

        def start(blk, buf):
            idx_v, rows_v, sem = buf
            base = base0 + blk * BLK
            pltpu.sync_copy(x_hbm.at[pl.ds((t0 * B + base) * W, BLK * W)],
                            idx_v)
            pltpu.make_async_copy(emb_hbm.at[idx_v], rows_v, sem).start()

        def finish(blk, buf):
            idx_v, rows_v, sem = buf
            pltpu.make_async_copy(emb_hbm.at[idx_v], rows_v, sem).wait()

            @pl.loop(0, BLK)
            def _(k):
                row = k * W
                accs = [rows_v[row, pl.ds(c * LANES, LANES)]
                        for c in range(NCHUNK)]
                for w in range(1, W):
                    accs = [accs[c] + rows_v[row + w, pl.ds(c * LANES, LANES)]
                            for c in range(NCHUNK)]
                for c in range(NCHUNK):
                    out_v[k, pl.ds(c * LANES, LANES)] = accs[c]

            pltpu.sync_copy(out_v, out_hbm.at[pl.ds(base0 + blk * BLK, BLK)])

        start(0, bufs[0])
        start(1, bufs[1])

        @pl.loop(0, nblk - 2, step=2)
        def _(g):
            for b in range(2):
                finish(g + b, bufs[b])
                start(g + b + 2, bufs[b])

        for b in range(2):
            finish(nblk - 2 + b, bufs[b])

    return gather_sum(xflat, embedding, dep)


K1 = EMB + HID + 1


def _lstm_chunk_body(tch, s_ref, h_ref, c_ref, wi, wf, wg, wo,
                     ho_ref, co_ref, xcat_ref, c_scr):
    xcat_ref[:, EMB:EMB + HID] = h_ref[...].astype(jnp.bfloat16)
    xcat_ref[:, EMB + HID:] = jnp.ones((B, 1), jnp.bfloat16)
    c_scr[...] = c_ref[...]

    def step(t, _):
        xcat_ref[:, :EMB] = s_ref[t].astype(jnp.bfloat16)
        xb = xcat_ref[...]
        dot = lambda a, b: jnp.dot(a, b, preferred_element_type=jnp.float32)
        sig = lambda z: 0.5 * jnp.tanh(0.5 * z) + 0.5
        ig = sig(dot(xb, wi[...]))
        fg = sig(dot(xb, wf[...]))
        gg = jnp.tanh(dot(xb, wg[...]))
        og = sig(dot(xb, wo[...]))
        c2 = fg * c_scr[...] + ig * gg
        c_scr[...] = c2
        xcat_ref[:, EMB:EMB + HID] = (og * jnp.tanh(c2)).astype(jnp.bfloat16)
        return 0

    lax.fori_loop(0, tch, step, 0)
    ho_ref[...] = xcat_ref[:, EMB:EMB + HID].astype(jnp.float32)
    co_ref[...] = c_scr[...]


def _lstm_chunk(tch, s, h, c, ws):
    return pl.pallas_call(
        functools.partial(_lstm_chunk_body, tch),
        out_shape=(jax.ShapeDtypeStruct((B, HID), jnp.float32),
                   jax.ShapeDtypeStruct((B, HID), jnp.float32)),
        scratch_shapes=[pltpu.VMEM((B, K1), jnp.bfloat16),
                        pltpu.VMEM((B, HID), jnp.float32)],
    )(s, h, c, *ws)


def _dense_softmax_body(h_ref, wd, bd, out_ref):
    logits = jnp.dot(h_ref[...], wd[...],
                     preferred_element_type=jnp.float32) + bd[...]
    m = jnp.max(logits, axis=-1, keepdims=True)
    e = jnp.exp(logits - m)
    out_ref[...] = e / jnp.sum(e, axis=-1, keepdims=True)


def kernel(x, embedding, W_lstm, U_lstm, b_lstm, W_dense, b_dense):
    wcat = jnp.concatenate(
        [W_lstm, U_lstm, b_lstm.reshape(1, 4 * HID)], axis=0
    ).astype(jnp.bfloat16)
    ws = [wcat[:, i * HID:(i + 1) * HID] for i in range(4)]

    h = jnp.zeros((B, HID), jnp.float32)
    c = jnp.zeros((B, HID), jnp.float32)
    xflat = x.transpose(1, 0, 2).reshape(T * B * W)
    zdep = jnp.zeros((8, 8), jnp.float32)
    offs = [sum(CHS[:k]) for k in range(NCH)]
    s_chunks = {}
    for k in range(2):
        s_chunks[k] = _sc_gather_sum(xflat, embedding, zdep, offs[k], CHS[k])
    for k in range(NCH):
        h, c = _lstm_chunk(CHS[k], s_chunks[k].reshape(CHS[k], B, EMB),
                           h, c, ws)
        nxt = k + 2
        if nxt < NCH:
            s_chunks[nxt] = _sc_gather_sum(xflat, embedding, h[:8, :8],
                                           offs[nxt], CHS[nxt])

    return pl.pallas_call(
        _dense_softmax_body,
        out_shape=jax.ShapeDtypeStruct((B, OUT), jnp.float32),
    )(h, W_dense, b_dense.reshape(1, OUT))


# speedup vs baseline: 1.5438x; 1.0270x over previous
import functools

import jax
import jax.numpy as jnp
from jax import lax
from jax.experimental import pallas as pl
from jax.experimental.pallas import tpu as pltpu
from jax.experimental.pallas import tpu_sc as plsc

EMB = 64
HID = 400
OUT = 1000
B, T, W = 1024, 50, 20

NW = 32
BLK = 32
LANES = 16
NCHUNK = EMB // LANES

CHS = (4, 6, 10, 14, 16)
NCH = len(CHS)


def _sc_gather_sum(xflat, embedding, dep, t0, tn):
    rows = tn * B
    rows_per_w = rows // NW
    nblk = rows_per_w // BLK
    assert nblk % 2 == 0
    mesh = plsc.VectorSubcoreMesh(core_axis_name="c", subcore_axis_name="s")

    @functools.partial(
        pl.kernel,
        out_type=jax.ShapeDtypeStruct((rows, EMB), jnp.float32),
        mesh=mesh,
        scratch_types=[
            pltpu.VMEM((BLK * W,), jnp.int32),
            pltpu.VMEM((BLK * W,), jnp.int32),
            pltpu.VMEM((BLK * W, EMB), jnp.float32),
            pltpu.VMEM((BLK * W, EMB), jnp.float32),
            pltpu.VMEM((BLK, EMB), jnp.float32),
            pltpu.SemaphoreType.DMA,
            pltpu.SemaphoreType.DMA,
        ],
        compiler_params=pltpu.CompilerParams(use_tc_tiling_on_sc=False),
    )
    def gather_sum(x_hbm, emb_hbm, dep_hbm, out_hbm, idx0, idx1,
                   rows0, rows1, out_v, sem0, sem1):
        del dep_hbm
        wid = lax.axis_index("s") * 2 + lax.axis_index("c")
        base0 = wid * rows_per_w
        bufs = ((idx0, rows0, sem0), (idx1, rows1, sem1))

        def start(blk, buf):
            idx_v, rows_v, sem = buf
            base = base0 + blk * BLK
            pltpu.sync_copy(x_hbm.at[pl.ds((t0 * B + base) * W, BLK * W)],
                            idx_v)
            pltpu.make_async_copy(emb_hbm.at[idx_v], rows_v, sem).start()

        def finish(blk, buf):
            idx_v, rows_v, sem = buf
            pltpu.make_async_copy(emb_hbm.at[idx_v], rows_v, sem).wait()

            @pl.loop(0, BLK)
            def _(k):
                row = k * W
                accs = [rows_v[row, pl.ds(c * LANES, LANES)]
                        for c in range(NCHUNK)]
                for w in range(1, W):
                    accs = [accs[c] + rows_v[row + w, pl.ds(c * LANES, LANES)]
                            for c in range(NCHUNK)]
                for c in range(NCHUNK):
                    out_v[k, pl.ds(c * LANES, LANES)] = accs[c]

            pltpu.sync_copy(out_v, out_hbm.at[pl.ds(base0 + blk * BLK, BLK)])

        start(0, bufs[0])
        start(1, bufs[1])

        @pl.loop(0, nblk - 2, step=2)
        def _(g):
            for b in range(2):
                finish(g + b, bufs[b])
                start(g + b + 2, bufs[b])

        for b in range(2):
            finish(nblk - 2 + b, bufs[b])

    return gather_sum(xflat, embedding, dep)


K1 = EMB + HID + 1


def _lstm_chunk_body(tch, s_ref, h_ref, c_ref, wi, wf, wg, wo,
                     ho_ref, co_ref, xcat_ref, c_scr):
    xcat_ref[:, EMB:EMB + HID] = h_ref[...].astype(jnp.bfloat16)
    xcat_ref[:, EMB + HID:] = jnp.ones((B, 1), jnp.bfloat16)
    c_scr[...] = c_ref[...]

    def one_step(t):
        xcat_ref[:, :EMB] = s_ref[t].astype(jnp.bfloat16)
        xb = xcat_ref[...]
        dot = lambda a, b: jnp.dot(a, b, preferred_element_type=jnp.float32)
        sig = lambda z: 0.5 * jnp.tanh(0.5 * z) + 0.5
        ig = sig(dot(xb, wi[...]))
        fg = sig(dot(xb, wf[...]))
        gg = jnp.tanh(dot(xb, wg[...]))
        og = sig(dot(xb, wo[...]))
        c2 = fg * c_scr[...] + ig * gg
        c_scr[...] = c2
        xcat_ref[:, EMB:EMB + HID] = (og * jnp.tanh(c2)).astype(jnp.bfloat16)

    def step2(t, _):
        one_step(2 * t)
        one_step(2 * t + 1)
        return 0

    lax.fori_loop(0, tch // 2, step2, 0)
    ho_ref[...] = xcat_ref[:, EMB:EMB + HID].astype(jnp.float32)
    co_ref[...] = c_scr[...]


def _lstm_chunk(tch, s, h, c, ws):
    return pl.pallas_call(
        functools.partial(_lstm_chunk_body, tch),
        out_shape=(jax.ShapeDtypeStruct((B, HID), jnp.float32),
                   jax.ShapeDtypeStruct((B, HID), jnp.float32)),
        scratch_shapes=[pltpu.VMEM((B, K1), jnp.bfloat16),
                        pltpu.VMEM((B, HID), jnp.float32)],
    )(s, h, c, *ws)


def _dense_softmax_body(h_ref, wd, bd, out_ref):
    logits = jnp.dot(h_ref[...], wd[...],
                     preferred_element_type=jnp.float32) + bd[...]
    m = jnp.max(logits, axis=-1, keepdims=True)
    e = jnp.exp(logits - m)
    out_ref[...] = e / jnp.sum(e, axis=-1, keepdims=True)


def kernel(x, embedding, W_lstm, U_lstm, b_lstm, W_dense, b_dense):
    wcat = jnp.concatenate(
        [W_lstm, U_lstm, b_lstm.reshape(1, 4 * HID)], axis=0
    ).astype(jnp.bfloat16)
    ws = [wcat[:, i * HID:(i + 1) * HID] for i in range(4)]

    h = jnp.zeros((B, HID), jnp.float32)
    c = jnp.zeros((B, HID), jnp.float32)
    xflat = x.transpose(1, 0, 2).reshape(T * B * W)
    zdep = jnp.zeros((8, 8), jnp.float32)
    offs = [sum(CHS[:k]) for k in range(NCH)]
    s_chunks = {}
    for k in range(2):
        s_chunks[k] = _sc_gather_sum(xflat, embedding, zdep, offs[k], CHS[k])
    for k in range(NCH):
        h, c = _lstm_chunk(CHS[k], s_chunks[k].reshape(CHS[k], B, EMB),
                           h, c, ws)
        nxt = k + 2
        if nxt < NCH:
            s_chunks[nxt] = _sc_gather_sum(xflat, embedding, h[:8, :8],
                                           offs[nxt], CHS[nxt])

    return pl.pallas_call(
        _dense_softmax_body,
        out_shape=jax.ShapeDtypeStruct((B, OUT), jnp.float32),
    )(h, W_dense, b_dense.reshape(1, OUT))
